# Initial kernel scaffold; baseline (speedup 1.0000x reference)
import jax
import jax.numpy as jnp
from jax.experimental import pallas as pl


def kernel(inputs, formal_charge, n_atoms, n_representations, n_molecules):
    raise NotImplementedError("write your pallas kernel here")


# speedup vs baseline: 7.0640x; 7.0640x over previous
import jax
import jax.numpy as jnp
from jax import lax
from jax.experimental import pallas as pl
from jax.experimental.pallas import tpu as pltpu
from jax.experimental.pallas import tpu_sc as plsc

_N_MOL = 16
_N_REP = 4
_N_ATOM = 512
_L = 16


def _sc_body(in_hbm, fc_hbm, out_hbm, buf_v, fc_v, out_v, sem_in, sem_fc):
    wid = lax.axis_index("s") * 2 + lax.axis_index("c")
    m = wid // 2
    half = wid % 2

    tok0 = m * (_N_REP * _N_ATOM)
    cp_in = pltpu.async_copy(
        in_hbm.at[pl.ds(tok0 * 3, _N_REP * _N_ATOM * 3)], buf_v, sem_in)
    cp_fc = pltpu.async_copy(
        fc_hbm.at[pl.ds(tok0, _N_REP * _N_ATOM)], fc_v, sem_fc)
    cp_in.wait()
    cp_fc.wait()

    iota3 = lax.iota(jnp.int32, _L) * 3

    fracs = []
    for r in range(_N_REP):
        base = r * _N_ATOM

        def seg_body(j, carry, base=base):
            sp, seos, sinvh, sfc = carry
            t0 = base + j * _L
            i0 = t0 * 3 + iota3
            p = plsc.load_gather(buf_v, [i0])
            e = plsc.load_gather(buf_v, [i0 + 1])
            h = plsc.load_gather(buf_v, [i0 + 2])
            fc = fc_v[pl.ds(t0, _L)]
            invh = 1.0 / h
            return (sp + p, seos + e * invh, sinvh + invh, sfc + fc)

        z = jnp.zeros((_L,), jnp.float32)
        sp, seos, sinvh, sfc = lax.fori_loop(
            0, _N_ATOM // _L, seg_body, (z, z, z, z))
        num = jnp.broadcast_to(jnp.sum(sp) - jnp.sum(sfc) - jnp.sum(seos), (_L,))
        den = jnp.broadcast_to(jnp.sum(sinvh), (_L,))
        fracs.append(num / den)

    a0 = half * (_N_ATOM // 2)

    def out_body(g, carry):
        a = a0 + g * _L
        acc = jnp.zeros((_L,), jnp.float32)
        for r in range(_N_REP):
            i0 = (r * _N_ATOM + a) * 3 + iota3
            p = plsc.load_gather(buf_v, [i0])
            e = plsc.load_gather(buf_v, [i0 + 1])
            h = plsc.load_gather(buf_v, [i0 + 2])
            acc = acc + (p - (e + fracs[r]) / h)
        out_v[pl.ds(g * _L, _L)] = acc * (1.0 / _N_REP)
        return carry

    lax.fori_loop(0, (_N_ATOM // 2) // _L, out_body, 0)
    pltpu.sync_copy(out_v, out_hbm.at[pl.ds(m * _N_ATOM + a0, _N_ATOM // 2)])


def kernel(inputs, formal_charge, n_atoms, n_representations, n_molecules):
    mesh = plsc.VectorSubcoreMesh(core_axis_name="c", subcore_axis_name="s")
    run = pl.kernel(
        _sc_body,
        out_type=jax.ShapeDtypeStruct((_N_MOL * _N_ATOM,), jnp.float32),
        mesh=mesh,
        compiler_params=pltpu.CompilerParams(needs_layout_passes=False),
        scratch_types=[
            pltpu.VMEM((_N_REP * _N_ATOM * 3,), jnp.float32),
            pltpu.VMEM((_N_REP * _N_ATOM,), jnp.float32),
            pltpu.VMEM((_N_ATOM // 2,), jnp.float32),
            pltpu.SemaphoreType.DMA,
            pltpu.SemaphoreType.DMA,
        ],
    )
    out = run(inputs.reshape(-1), formal_charge)
    return out.reshape(-1, 1)
